# Initial kernel scaffold; baseline (speedup 1.0000x reference)
import jax
import jax.numpy as jnp
from jax.experimental import pallas as pl


def kernel(V, U, centers, contexts_negs):
    raise NotImplementedError("write your pallas kernel here")


# speedup vs baseline: 11.6969x; 11.6969x over previous
import functools

import jax
import jax.numpy as jnp
from jax import lax
from jax.experimental import pallas as pl
from jax.experimental.pallas import tpu as pltpu
from jax.experimental.pallas import tpu_sc as plsc

_NC = 2
_NS = 16
_NW = _NC * _NS
_LANES = 16


def _mm_body(v_ref, u_ref, g_ref):
    g_ref[...] = lax.dot_general(
        v_ref[...], u_ref[...], (((1,), (1,)), ((), ())),
        preferred_element_type=jnp.float32,
        precision=lax.Precision.HIGHEST)


@functools.lru_cache(maxsize=None)
def _make_sc_gather(B, K1, TV):
    bpw = B // _NW
    E = bpw * K1
    n_build = bpw // _LANES
    n_chunk = E // 128
    n_red = E // _LANES
    mesh = plsc.VectorSubcoreMesh(core_axis_name="c", subcore_axis_name="s")

    @functools.partial(
        pl.kernel, mesh=mesh,
        out_type=(jax.ShapeDtypeStruct((_NW, _LANES), jnp.float32),
                  jax.ShapeDtypeStruct((_NW, _LANES), jnp.float32)),
        scratch_types=[
            pltpu.VMEM((bpw,), jnp.int32),
            pltpu.VMEM((E,), jnp.int32),
            pltpu.VMEM((E,), jnp.int32),
            pltpu.VMEM((E,), jnp.float32),
            pltpu.VMEM((_LANES,), jnp.float32),
            pltpu.VMEM((_LANES,), jnp.float32),
            pltpu.SemaphoreType.DMA,
        ])
    def sc_fn(g_hbm, cen_hbm, cn_hbm, esum_hbm, lsum_hbm,
              cen_v, cn_v, idx_v, val_v, es_v, ls_v, sem):
        wid = lax.axis_index("s") * _NC + lax.axis_index("c")
        base = wid * bpw
        pltpu.sync_copy(cen_hbm.at[pl.ds(base, bpw)], cen_v)
        for k in range(K1):
            pltpu.sync_copy(cn_hbm.at[pl.ds(k * B + base, bpw)],
                            cn_v.at[pl.ds(k * bpw, bpw)])

        for i in range(n_build):
            c16 = cen_v[pl.ds(i * _LANES, _LANES)]
            for k in range(K1):
                nk = cn_v[pl.ds(k * bpw + i * _LANES, _LANES)]
                idx_v[pl.ds(k * bpw + i * _LANES, _LANES)] = c16 * TV + nk

        copies = [
            pltpu.async_copy(
                g_hbm.at[idx_v.at[pl.ds(j * 128, 128)]],
                val_v.at[pl.ds(j * 128, 128)], sem)
            for j in range(n_chunk)
        ]
        for cp in copies:
            cp.wait()

        acc = jnp.zeros((_LANES,), jnp.float32)
        acc0 = jnp.zeros((_LANES,), jnp.float32)
        for i in range(n_red):
            x = val_v[pl.ds(i * _LANES, _LANES)]
            acc = acc + jnp.exp(x)
            if i < n_build:
                acc0 = acc0 + x
        es_v[...] = acc
        ls_v[...] = acc0
        pltpu.sync_copy(es_v, esum_hbm.at[wid])
        pltpu.sync_copy(ls_v, lsum_hbm.at[wid])

    return sc_fn


@functools.lru_cache(maxsize=None)
def _make_finalize(B):
    def _fin_body(es_ref, ls_ref, out_ref):
        s = jnp.sum(es_ref[...])
        l0 = jnp.sum(ls_ref[...])
        out_ref[...] = jnp.reshape(float(B) * jnp.log(s) - l0, (1, 1))

    return pl.pallas_call(
        _fin_body,
        out_shape=jax.ShapeDtypeStruct((1, 1), jnp.float32))


def kernel(V, U, centers, contexts_negs):
    voc, d = V.shape
    B = centers.shape[0]
    K1 = contexts_negs.shape[1]
    TV = 1024
    Vp = jnp.pad(V, ((0, TV - voc), (0, 0)))
    Up = jnp.pad(U, ((0, TV - voc), (0, 0)))
    G = pl.pallas_call(
        _mm_body,
        out_shape=jax.ShapeDtypeStruct((TV, TV), jnp.float32))(Vp, Up)
    g_flat = G.reshape(TV * TV)
    cn_t = contexts_negs.T.reshape(K1 * B)
    esum, lsum = _make_sc_gather(B, K1, TV)(g_flat, centers, cn_t)
    loss = _make_finalize(B)(esum, lsum)
    return loss[0, 0]
